# Initial kernel scaffold; baseline (speedup 1.0000x reference)
import jax
import jax.numpy as jnp
from jax.experimental import pallas as pl


def kernel(x, pos_table):
    raise NotImplementedError("write your pallas kernel here")


# speedup vs baseline: 3.5234x; 3.5234x over previous
import jax
import jax.numpy as jnp
from jax.experimental import pallas as pl
from jax.experimental.pallas import tpu as pltpu


def _add_kernel(x_ref, pos_ref, o_ref):
    o_ref[...] = x_ref[...] + pos_ref[...]


def kernel(x, pos_table):
    batch, max_len, dim = x.shape
    x2 = x.reshape(batch * max_len, dim)
    pos = pos_table[:max_len]

    out = pl.pallas_call(
        _add_kernel,
        grid=(batch,),
        in_specs=[
            pl.BlockSpec((max_len, dim), lambda i: (i, 0)),
            pl.BlockSpec((max_len, dim), lambda i: (0, 0)),
        ],
        out_specs=pl.BlockSpec((max_len, dim), lambda i: (i, 0)),
        out_shape=jax.ShapeDtypeStruct((batch * max_len, dim), x.dtype),
        compiler_params=pltpu.CompilerParams(
            dimension_semantics=("arbitrary",),
        ),
    )(x2, pos)
    return out.reshape(batch, max_len, dim)


# speedup vs baseline: 3.6797x; 1.0444x over previous
import jax
import jax.numpy as jnp
from jax.experimental import pallas as pl
from jax.experimental.pallas import tpu as pltpu


def _add_kernel(x_ref, pos_ref, o_ref):
    o_ref[...] = x_ref[...] + pos_ref[...][None]


_BB = 4


def kernel(x, pos_table):
    batch, max_len, dim = x.shape
    pos = pos_table[:max_len]

    out = pl.pallas_call(
        _add_kernel,
        grid=(batch // _BB,),
        in_specs=[
            pl.BlockSpec((_BB, max_len, dim), lambda i: (i, 0, 0)),
            pl.BlockSpec((max_len, dim), lambda i: (0, 0)),
        ],
        out_specs=pl.BlockSpec((_BB, max_len, dim), lambda i: (i, 0, 0)),
        out_shape=jax.ShapeDtypeStruct((batch, max_len, dim), x.dtype),
        compiler_params=pltpu.CompilerParams(
            dimension_semantics=("arbitrary",),
        ),
    )(x, pos)
    return out
